# Initial kernel scaffold; baseline (speedup 1.0000x reference)
import jax
import jax.numpy as jnp
from jax.experimental import pallas as pl


def kernel(current_band_0, current_band_1, current_band_2, history_band_0, history_band_1, history_band_2, Wq_t, Wk_t, Wv_t, Wq_n, Wk_n, Wv_n, Wv_w, fus_W1, fus_b1, fus_W2, fus_b2, current_step):
    raise NotImplementedError("write your pallas kernel here")


# speedup vs baseline: 3.3196x; 3.3196x over previous
import jax
import jax.numpy as jnp
from jax.experimental import pallas as pl
from jax.experimental.pallas import tpu as pltpu

_H, _W, _T, _D, _A = 32, 32, 32, 256, 512
_N = _H * _W
_K = _T * _N
_TOPK_T = 4
_DECAY = 0.95
_LAYER_RANGE = 5
_WH_THRESH = 0.0
_WH_MAXCONN = 16
_TAU = 1.0
_NEG = -1e9

_bf16 = jnp.bfloat16
_f32 = jnp.float32


def _bdot(a, b):
    return jnp.dot(a.astype(_bf16), b.astype(_bf16),
                   preferred_element_type=_f32)


def _bdot_t(a, b):
    return jax.lax.dot_general(a.astype(_bf16), b.astype(_bf16),
                               (((1,), (1,)), ((), ())),
                               preferred_element_type=_f32)


def _norm_kernel(x_ref, o_ref):
    x = x_ref[...]
    n = jnp.sqrt(jnp.sum(x * x, axis=-1, keepdims=True))
    o_ref[...] = (x / (n + 1e-6)).astype(_bf16)


def _small_kernel(cb0_ref, cb1_ref, hb0_ref, hb2_ref,
                  wqt_ref, wkt_ref, wqn_ref, wkn_ref,
                  aggt_ref, aggn_ref):
    scale = 1.0 / jnp.sqrt(jnp.float32(_A))
    cb0 = cb0_ref[...]
    cb1 = cb1_ref[...]
    qt = _bdot(cb0, wqt_ref[...])
    qt_r = qt.astype(_bf16).astype(_f32)
    wkt = wkt_ref[...]
    cols = []
    logd = jnp.log(jnp.float32(_DECAY))
    for t in range(_T):
        kt = _bdot(hb0_ref[t], wkt)
        kt_r = kt.astype(_bf16).astype(_f32)
        s = jnp.sum(qt_r * kt_r, axis=-1) * scale + logd * (_T - t)
        cols.append(s[:, None])
    scores = jnp.concatenate(cols, axis=1)
    work = scores
    m1 = None
    vk = None
    for i in range(_TOPK_T):
        vk = jnp.max(work, axis=1, keepdims=True)
        if i == 0:
            m1 = vk
        work = jnp.where(work == vk, -jnp.inf, work)
    e = jnp.where(scores >= vk, jnp.exp((scores - m1) / _TAU), 0.0)
    wt = e / jnp.sum(e, axis=1, keepdims=True)
    agg = wt[:, 0][:, None] * hb0_ref[0]
    for t in range(1, _T):
        agg = agg + wt[:, t][:, None] * hb0_ref[t]
    aggt_ref[...] = agg
    mn = _bdot_t(wqn_ref[...], wkn_ref[...])
    qkn = _bdot(cb1, mn)
    ncols = []
    for t in range(_LAYER_RANGE):
        s = jnp.sum(hb2_ref[t] * qkn, axis=-1) * scale
        ncols.append(s[:, None])
    sn = jnp.concatenate(ncols, axis=1)
    mx = jnp.max(sn, axis=1, keepdims=True)
    en = jnp.exp((sn - mx) / _TAU)
    wn = en / jnp.sum(en, axis=1, keepdims=True)
    aggn = wn[:, 0][:, None] * hb2_ref[0]
    for t in range(1, _LAYER_RANGE):
        aggn = aggn + wn[:, t][:, None] * hb2_ref[t]
    aggn_ref[...] = aggn


def _sim_kernel(cb1_ref, kn_ref, sim_ref):
    q = cb1_ref[...]
    qn = q / (jnp.sqrt(jnp.sum(q * q, axis=-1, keepdims=True)) + 1e-6)
    s = jax.lax.dot_general(qn.astype(_bf16), kn_ref[...],
                            (((1,), (1,)), ((), ())),
                            preferred_element_type=_f32)
    sim_ref[...] = jnp.where(s >= _WH_THRESH, s, _NEG)


def _topk_kernel(sim_ref, w_ref, scratch_ref):
    scratch_ref[...] = sim_ref[...]
    m1 = None
    vk = None
    for i in range(_WH_MAXCONN):
        vk = jnp.max(scratch_ref[...], axis=1, keepdims=True)
        if i == 0:
            m1 = vk
        scratch_ref[...] = jnp.where(scratch_ref[...] == vk, -jnp.inf,
                                     scratch_ref[...])
    sim = sim_ref[...]
    sel = (sim >= vk) & (sim > -1e8)
    e = jnp.where(sel, jnp.exp((sim - m1) / _TAU), 0.0)
    w_ref[...] = e / (jnp.sum(e, axis=1, keepdims=True) + 1e-9)


def _agg_kernel(w_ref, hb0_ref, out_ref, acc_ref):
    j = pl.program_id(0)
    i = pl.program_id(1)
    part = _bdot(w_ref[...], hb0_ref[...])
    sl = pl.ds(i * 128, 128)

    @pl.when(j == 0)
    def _():
        acc_ref[sl, :] = part

    @pl.when(j > 0)
    def _():
        acc_ref[sl, :] = acc_ref[sl, :] + part

    @pl.when(j == 15)
    def _():
        out_ref[...] = acc_ref[sl, :]


def _fuse_kernel(aggt_ref, aggn_ref, aggw_ref, wvt_ref, wvn_ref, wvw_ref,
                 w1a_ref, w1b_ref, w1c_ref, b1_ref, w2_ref, b2_ref, out_ref):
    ot = _bdot(aggt_ref[...], wvt_ref[...])
    on = _bdot(aggn_ref[...], wvn_ref[...])
    ow = _bdot(aggw_ref[...], wvw_ref[...])
    h = (_bdot(ot, w1a_ref[...]) + _bdot(on, w1b_ref[...])
         + _bdot(ow, w1c_ref[...]) + b1_ref[...])
    h = jnp.maximum(h, 0.0)
    out_ref[...] = _bdot(h, w2_ref[...]) + b2_ref[...]


def kernel(current_band_0, current_band_1, current_band_2,
           history_band_0, history_band_1, history_band_2,
           Wq_t, Wk_t, Wv_t, Wq_n, Wk_n, Wv_n, Wv_w,
           fus_W1, fus_b1, fus_W2, fus_b2, current_step):
    f32 = jnp.float32
    cb0f = current_band_0.reshape(_N, _D)
    cb1f = current_band_1.reshape(_N, _D)
    hb0_t = history_band_0.reshape(_T, _N, _D)
    hb0f = history_band_0.reshape(_K, _D)
    hb1f = history_band_1.reshape(_K, _D)
    hb2_5 = history_band_2[_T - _LAYER_RANGE:].reshape(_LAYER_RANGE, _N, _D)

    kn = pl.pallas_call(
        _norm_kernel,
        grid=(16,),
        in_specs=[pl.BlockSpec((2048, _D), lambda j: (j, 0))],
        out_specs=pl.BlockSpec((2048, _D), lambda j: (j, 0)),
        out_shape=jax.ShapeDtypeStruct((_K, _D), _bf16),
    )(hb1f)

    full = lambda shape: pl.BlockSpec(shape, lambda: tuple(0 for _ in shape))
    _PB = 128
    aggt, aggn = pl.pallas_call(
        _small_kernel,
        grid=(_N // _PB,),
        in_specs=[pl.BlockSpec((_PB, _D), lambda i: (i, 0)),
                  pl.BlockSpec((_PB, _D), lambda i: (i, 0)),
                  pl.BlockSpec((_T, _PB, _D), lambda i: (0, i, 0)),
                  pl.BlockSpec((_LAYER_RANGE, _PB, _D), lambda i: (0, i, 0)),
                  pl.BlockSpec((_D, _A), lambda i: (0, 0)),
                  pl.BlockSpec((_D, _A), lambda i: (0, 0)),
                  pl.BlockSpec((_D, _A), lambda i: (0, 0)),
                  pl.BlockSpec((_D, _A), lambda i: (0, 0))],
        out_specs=[pl.BlockSpec((_PB, _D), lambda i: (i, 0)),
                   pl.BlockSpec((_PB, _D), lambda i: (i, 0))],
        out_shape=[jax.ShapeDtypeStruct((_N, _D), f32),
                   jax.ShapeDtypeStruct((_N, _D), f32)],
    )(cb0f, cb1f, hb0_t, hb2_5, Wq_t, Wk_t, Wq_n, Wk_n)

    sim = pl.pallas_call(
        _sim_kernel,
        grid=(8, 16),
        in_specs=[pl.BlockSpec((128, _D), lambda i, j: (i, 0)),
                  pl.BlockSpec((2048, _D), lambda i, j: (j, 0))],
        out_specs=pl.BlockSpec((128, 2048), lambda i, j: (i, j)),
        out_shape=jax.ShapeDtypeStruct((_N, _K), f32),
    )(cb1f, kn)

    wmat = pl.pallas_call(
        _topk_kernel,
        grid=(32,),
        in_specs=[pl.BlockSpec((32, _K), lambda i: (i, 0))],
        out_specs=pl.BlockSpec((32, _K), lambda i: (i, 0)),
        out_shape=jax.ShapeDtypeStruct((_N, _K), f32),
        scratch_shapes=[pltpu.VMEM((32, _K), f32)],
    )(sim)

    aggw = pl.pallas_call(
        _agg_kernel,
        grid=(16, 8),
        in_specs=[pl.BlockSpec((128, 2048), lambda j, i: (i, j)),
                  pl.BlockSpec((2048, _D), lambda j, i: (j, 0))],
        out_specs=pl.BlockSpec((128, _D), lambda j, i: (i, 0)),
        out_shape=jax.ShapeDtypeStruct((_N, _D), f32),
        scratch_shapes=[pltpu.VMEM((_N, _D), f32)],
    )(wmat, hb0f)

    out = pl.pallas_call(
        _fuse_kernel,
        grid=(),
        in_specs=[full((_N, _D)), full((_N, _D)), full((_N, _D)),
                  full((_D, _D)), full((_D, _D)), full((_D, _D)),
                  full((_D, _A)), full((_D, _A)), full((_D, _A)),
                  full((1, _A)), full((_A, _A)), full((1, _A))],
        out_specs=full((_N, _A)),
        out_shape=jax.ShapeDtypeStruct((_N, _A), f32),
    )(aggt, aggn, aggw, Wv_t, Wv_n, Wv_w,
      fus_W1[:_D], fus_W1[_D:2 * _D], fus_W1[2 * _D:],
      fus_b1.reshape(1, _A), fus_W2, fus_b2.reshape(1, _A))

    return out.reshape(_H, _W, _A)


# speedup vs baseline: 3.6041x; 1.0857x over previous
import jax
import jax.numpy as jnp
from jax.experimental import pallas as pl
from jax.experimental.pallas import tpu as pltpu

_H, _W, _T, _D, _A = 32, 32, 32, 256, 512
_N = _H * _W
_K = _T * _N
_TOPK_T = 4
_DECAY = 0.95
_LAYER_RANGE = 5
_WH_THRESH = 0.0
_WH_MAXCONN = 16
_TAU = 1.0
_NEG = -1e9

_bf16 = jnp.bfloat16
_f32 = jnp.float32


def _bdot(a, b):
    return jnp.dot(a.astype(_bf16), b.astype(_bf16),
                   preferred_element_type=_f32)


def _bdot_t(a, b):
    return jax.lax.dot_general(a.astype(_bf16), b.astype(_bf16),
                               (((1,), (1,)), ((), ())),
                               preferred_element_type=_f32)


def _norm_kernel(x_ref, o_ref):
    x = x_ref[...]
    n = jnp.sqrt(jnp.sum(x * x, axis=-1, keepdims=True))
    o_ref[...] = (x / (n + 1e-6)).astype(_bf16)


def _small_kernel(cb0_ref, cb1_ref, hb0_ref, hb2_ref,
                  wqt_ref, wkt_ref, wqn_ref, wkn_ref,
                  aggt_ref, aggn_ref):
    scale = 1.0 / jnp.sqrt(jnp.float32(_A))
    cb0 = cb0_ref[...]
    cb1 = cb1_ref[...]
    qt = _bdot(cb0, wqt_ref[...])
    qt_r = qt.astype(_bf16).astype(_f32)
    wkt = wkt_ref[...]
    cols = []
    logd = jnp.log(jnp.float32(_DECAY))
    for t in range(_T):
        kt = _bdot(hb0_ref[t], wkt)
        kt_r = kt.astype(_bf16).astype(_f32)
        s = jnp.sum(qt_r * kt_r, axis=-1) * scale + logd * (_T - t)
        cols.append(s[:, None])
    scores = jnp.concatenate(cols, axis=1)
    work = scores
    m1 = None
    vk = None
    for i in range(_TOPK_T):
        vk = jnp.max(work, axis=1, keepdims=True)
        if i == 0:
            m1 = vk
        work = jnp.where(work == vk, -jnp.inf, work)
    e = jnp.where(scores >= vk, jnp.exp((scores - m1) / _TAU), 0.0)
    wt = e / jnp.sum(e, axis=1, keepdims=True)
    agg = wt[:, 0][:, None] * hb0_ref[0]
    for t in range(1, _T):
        agg = agg + wt[:, t][:, None] * hb0_ref[t]
    aggt_ref[...] = agg
    mn = _bdot_t(wqn_ref[...], wkn_ref[...])
    qkn = _bdot(cb1, mn)
    ncols = []
    for t in range(_LAYER_RANGE):
        s = jnp.sum(hb2_ref[t] * qkn, axis=-1) * scale
        ncols.append(s[:, None])
    sn = jnp.concatenate(ncols, axis=1)
    mx = jnp.max(sn, axis=1, keepdims=True)
    en = jnp.exp((sn - mx) / _TAU)
    wn = en / jnp.sum(en, axis=1, keepdims=True)
    aggn = wn[:, 0][:, None] * hb2_ref[0]
    for t in range(1, _LAYER_RANGE):
        aggn = aggn + wn[:, t][:, None] * hb2_ref[t]
    aggn_ref[...] = aggn


_QB = 128
_CK = 2048
_NJ = _K // _CK
_LCH = 128
_NCH = _CK // _LCH
_TOTCH = _K // _LCH
_TOP_PER_CH = 4


def _worm_kernel(cb1_ref, kn_ref, hb0_ref, out_ref, simbuf, cand_ref, v16_ref):
    j = pl.program_id(1)

    @pl.when(j < _NJ)
    def _sim_phase():
        q = cb1_ref[...]
        qn = (q / (jnp.sqrt(jnp.sum(q * q, axis=-1, keepdims=True)) + 1e-6)
              ).astype(_bf16)
        s = jax.lax.dot_general(qn, kn_ref[...], (((1,), (1,)), ((), ())),
                                preferred_element_type=_f32)
        sm = jnp.where(s >= 0.0, s, _NEG)
        simbuf[:, pl.ds(pl.multiple_of(j * _CK, _CK), _CK)] = sm
        tops = []
        for c in range(_NCH):
            blk = sm[:, c * _LCH:(c + 1) * _LCH]
            m = jnp.max(blk, axis=1, keepdims=True)
            tops.append(m)
            for _ in range(1, _TOP_PER_CH):
                blk = jnp.where(blk < m, blk, -jnp.inf)
                m = jnp.max(blk, axis=1, keepdims=True)
                tops.append(m)
        pad = jnp.full((_QB, _LCH - _NCH * _TOP_PER_CH), -jnp.inf, dtype=_f32)
        block = jnp.concatenate(tops + [pad], axis=1)
        cand_ref[:, pl.ds(pl.multiple_of(j * _LCH, _LCH), _LCH)] = block

    @pl.when(j == _NJ)
    def _select_phase():
        cand = cand_ref[...]
        m1 = jnp.max(cand, axis=1, keepdims=True)
        v = m1
        for _ in range(15):
            v = jnp.max(jnp.where(cand < v, cand, -jnp.inf), axis=1,
                        keepdims=True)
        cnt = jnp.zeros((_QB, 1), dtype=_f32)
        for t in range(_NJ):
            st = simbuf[:, t * _CK:(t + 1) * _CK]
            cnt = cnt + jnp.sum((st >= v).astype(_f32), axis=1, keepdims=True)
        bad = jnp.any((v[:, 0] > -1e8) & (cnt[:, 0] > 16.5))

        @pl.when(bad)
        def _exact_fallback():
            vv = m1
            for _ in range(15):
                nxt = None
                for t in range(_NJ):
                    st = simbuf[:, t * _CK:(t + 1) * _CK]
                    pm = jnp.max(jnp.where(st < vv, st, -jnp.inf), axis=1,
                                 keepdims=True)
                    nxt = pm if nxt is None else jnp.maximum(nxt, pm)
                vv = nxt
            v16_ref[...] = jnp.broadcast_to(vv, (_QB, _LCH))

        @pl.when(jnp.logical_not(bad))
        def _fast():
            v16_ref[...] = jnp.broadcast_to(v, (_QB, _LCH))

        v16 = v16_ref[...][:, 0:1]
        den = jnp.zeros((_QB, 1), dtype=_f32)
        for t in range(_NJ):
            st = simbuf[:, t * _CK:(t + 1) * _CK]
            sel = (st >= v16) & (st > -1e8)
            e = jnp.where(sel, jnp.exp(st - m1), 0.0)
            simbuf[:, pl.ds(t * _CK, _CK)] = e
            den = den + jnp.sum(e, axis=1, keepdims=True)
        acc = jnp.zeros((_QB, _D), dtype=_f32)
        for t in range(_NJ):
            e = simbuf[:, t * _CK:(t + 1) * _CK]
            hv = hb0_ref[pl.ds(t * _CK, _CK), :]
            acc = acc + jnp.dot(e.astype(_bf16), hv,
                                preferred_element_type=_f32)
        out_ref[...] = acc / (den + 1e-9)


def _fuse_kernel(aggt_ref, aggn_ref, aggw_ref, wvt_ref, wvn_ref, wvw_ref,
                 w1a_ref, w1b_ref, w1c_ref, b1_ref, w2_ref, b2_ref, out_ref):
    ot = _bdot(aggt_ref[...], wvt_ref[...])
    on = _bdot(aggn_ref[...], wvn_ref[...])
    ow = _bdot(aggw_ref[...], wvw_ref[...])
    h = (_bdot(ot, w1a_ref[...]) + _bdot(on, w1b_ref[...])
         + _bdot(ow, w1c_ref[...]) + b1_ref[...])
    h = jnp.maximum(h, 0.0)
    out_ref[...] = _bdot(h, w2_ref[...]) + b2_ref[...]


def kernel(current_band_0, current_band_1, current_band_2,
           history_band_0, history_band_1, history_band_2,
           Wq_t, Wk_t, Wv_t, Wq_n, Wk_n, Wv_n, Wv_w,
           fus_W1, fus_b1, fus_W2, fus_b2, current_step):
    f32 = jnp.float32
    cb0f = current_band_0.reshape(_N, _D)
    cb1f = current_band_1.reshape(_N, _D)
    hb0_t = history_band_0.reshape(_T, _N, _D)
    hb0_bf = history_band_0.reshape(_K, _D).astype(_bf16)
    hb1f = history_band_1.reshape(_K, _D)
    hb2_5 = history_band_2[_T - _LAYER_RANGE:].reshape(_LAYER_RANGE, _N, _D)

    kn = pl.pallas_call(
        _norm_kernel,
        grid=(16,),
        in_specs=[pl.BlockSpec((2048, _D), lambda j: (j, 0))],
        out_specs=pl.BlockSpec((2048, _D), lambda j: (j, 0)),
        out_shape=jax.ShapeDtypeStruct((_K, _D), _bf16),
    )(hb1f)

    full = lambda shape: pl.BlockSpec(shape, lambda: tuple(0 for _ in shape))
    _PB = 128
    aggt, aggn = pl.pallas_call(
        _small_kernel,
        grid=(_N // _PB,),
        in_specs=[pl.BlockSpec((_PB, _D), lambda i: (i, 0)),
                  pl.BlockSpec((_PB, _D), lambda i: (i, 0)),
                  pl.BlockSpec((_T, _PB, _D), lambda i: (0, i, 0)),
                  pl.BlockSpec((_LAYER_RANGE, _PB, _D), lambda i: (0, i, 0)),
                  pl.BlockSpec((_D, _A), lambda i: (0, 0)),
                  pl.BlockSpec((_D, _A), lambda i: (0, 0)),
                  pl.BlockSpec((_D, _A), lambda i: (0, 0)),
                  pl.BlockSpec((_D, _A), lambda i: (0, 0))],
        out_specs=[pl.BlockSpec((_PB, _D), lambda i: (i, 0)),
                   pl.BlockSpec((_PB, _D), lambda i: (i, 0))],
        out_shape=[jax.ShapeDtypeStruct((_N, _D), f32),
                   jax.ShapeDtypeStruct((_N, _D), f32)],
    )(cb0f, cb1f, hb0_t, hb2_5, Wq_t, Wk_t, Wq_n, Wk_n)

    aggw = pl.pallas_call(
        _worm_kernel,
        grid=(_N // _QB, _NJ + 1),
        in_specs=[pl.BlockSpec((_QB, _D), lambda i, j: (i, 0)),
                  pl.BlockSpec((_CK, _D), lambda i, j: (jnp.minimum(j, _NJ - 1), 0)),
                  pl.BlockSpec((_K, _D), lambda i, j: (0, 0))],
        out_specs=pl.BlockSpec((_QB, _D), lambda i, j: (i, 0)),
        out_shape=jax.ShapeDtypeStruct((_N, _D), f32),
        scratch_shapes=[pltpu.VMEM((_QB, _K), f32),
                        pltpu.VMEM((_QB, _NJ * _LCH), f32),
                        pltpu.VMEM((_QB, _LCH), f32)],
    )(cb1f, kn, hb0_bf)

    out = pl.pallas_call(
        _fuse_kernel,
        grid=(),
        in_specs=[full((_N, _D)), full((_N, _D)), full((_N, _D)),
                  full((_D, _D)), full((_D, _D)), full((_D, _D)),
                  full((_D, _A)), full((_D, _A)), full((_D, _A)),
                  full((1, _A)), full((_A, _A)), full((1, _A))],
        out_specs=full((_N, _A)),
        out_shape=jax.ShapeDtypeStruct((_N, _A), f32),
    )(aggt, aggn, aggw, Wv_t, Wv_n, Wv_w,
      fus_W1[:_D], fus_W1[_D:2 * _D], fus_W1[2 * _D:],
      fus_b1.reshape(1, _A), fus_W2, fus_b2.reshape(1, _A))

    return out.reshape(_H, _W, _A)


# speedup vs baseline: 4.2448x; 1.1778x over previous
import jax
import jax.numpy as jnp
from jax.experimental import pallas as pl
from jax.experimental.pallas import tpu as pltpu

_H, _W, _T, _D, _A = 32, 32, 32, 256, 512
_N = _H * _W
_K = _T * _N
_TOPK_T = 4
_DECAY = 0.95
_LAYER_RANGE = 5
_WH_THRESH = 0.0
_WH_MAXCONN = 16
_TAU = 1.0
_NEG = -1e9

_bf16 = jnp.bfloat16
_f32 = jnp.float32


def _bdot(a, b):
    return jnp.dot(a.astype(_bf16), b.astype(_bf16),
                   preferred_element_type=_f32)


def _bdot_t(a, b):
    return jax.lax.dot_general(a.astype(_bf16), b.astype(_bf16),
                               (((1,), (1,)), ((), ())),
                               preferred_element_type=_f32)


def _norm_kernel(x_ref, o_ref):
    x = x_ref[...]
    n = jnp.sqrt(jnp.sum(x * x, axis=-1, keepdims=True))
    o_ref[...] = (x / (n + 1e-6)).astype(_bf16)


def _small_kernel(cb0_ref, cb1_ref, hb0_ref, hb2_ref,
                  wqt_ref, wkt_ref, wqn_ref, wkn_ref,
                  aggt_ref, aggn_ref):
    scale = 1.0 / jnp.sqrt(jnp.float32(_A))
    cb0 = cb0_ref[...]
    cb1 = cb1_ref[...]
    qt = _bdot(cb0, wqt_ref[...])
    qt_r = qt.astype(_bf16).astype(_f32)
    pb = cb0.shape[0]
    hb0_all = hb0_ref[...].reshape(_T * pb, _D)
    kt_all = _bdot(hb0_all, wkt_ref[...])
    kt_all = kt_all.astype(_bf16).astype(_f32)
    cols = []
    logd = jnp.log(jnp.float32(_DECAY))
    for t in range(_T):
        kt_r = kt_all[t * pb:(t + 1) * pb]
        s = jnp.sum(qt_r * kt_r, axis=-1) * scale + logd * (_T - t)
        cols.append(s[:, None])
    scores = jnp.concatenate(cols, axis=1)
    work = scores
    m1 = None
    vk = None
    for i in range(_TOPK_T):
        vk = jnp.max(work, axis=1, keepdims=True)
        if i == 0:
            m1 = vk
        work = jnp.where(work == vk, -jnp.inf, work)
    e = jnp.where(scores >= vk, jnp.exp((scores - m1) / _TAU), 0.0)
    wt = e / jnp.sum(e, axis=1, keepdims=True)
    agg = wt[:, 0][:, None] * hb0_ref[0]
    for t in range(1, _T):
        agg = agg + wt[:, t][:, None] * hb0_ref[t]
    aggt_ref[...] = agg
    mn = _bdot_t(wqn_ref[...], wkn_ref[...])
    qkn = _bdot(cb1, mn)
    ncols = []
    for t in range(_LAYER_RANGE):
        s = jnp.sum(hb2_ref[t] * qkn, axis=-1) * scale
        ncols.append(s[:, None])
    sn = jnp.concatenate(ncols, axis=1)
    mx = jnp.max(sn, axis=1, keepdims=True)
    en = jnp.exp((sn - mx) / _TAU)
    wn = en / jnp.sum(en, axis=1, keepdims=True)
    aggn = wn[:, 0][:, None] * hb2_ref[0]
    for t in range(1, _LAYER_RANGE):
        aggn = aggn + wn[:, t][:, None] * hb2_ref[t]
    aggn_ref[...] = aggn


_QB = 256
_CK = 4096
_NJ = _K // _CK
_LCH = 128
_NCH = _CK // _LCH
_TOTCH = _K // _LCH
_TOP_PER_CH = 4


def _worm_kernel(cb1_ref, kn_ref, hb0_ref, out_ref, simbuf, cand_ref, v16_ref,
                 den_ref, acc_ref):
    j = pl.program_id(1)

    @pl.when(j < _NJ)
    def _sim_phase():
        q = cb1_ref[...]
        qn = (q / (jnp.sqrt(jnp.sum(q * q, axis=-1, keepdims=True)) + 1e-6)
              ).astype(_bf16)
        s = jax.lax.dot_general(qn, kn_ref[...], (((1,), (1,)), ((), ())),
                                preferred_element_type=_f32)
        sm = jnp.where(s >= 0.0, s, _NEG)
        simbuf[:, pl.ds(pl.multiple_of(j * _CK, _CK), _CK)] = sm
        tops = []
        for c in range(_NCH):
            blk = sm[:, c * _LCH:(c + 1) * _LCH]
            m = jnp.max(blk, axis=1, keepdims=True)
            tops.append(m)
            for _ in range(1, _TOP_PER_CH):
                blk = jnp.where(blk < m, blk, -jnp.inf)
                m = jnp.max(blk, axis=1, keepdims=True)
                tops.append(m)
        npad = _NCH * _TOP_PER_CH
        if npad < _LCH:
            tops.append(jnp.full((_QB, _LCH - npad), -jnp.inf, dtype=_f32))
        block = jnp.concatenate(tops, axis=1)
        nb = block.shape[1]
        cand_ref[:, pl.ds(pl.multiple_of(j * nb, _LCH), nb)] = block

    @pl.when(j == _NJ)
    def _select_phase():
        cand = cand_ref[...]
        m1 = jnp.max(cand, axis=1, keepdims=True)
        v = m1
        for _ in range(15):
            v = jnp.max(jnp.where(cand < v, cand, -jnp.inf), axis=1,
                        keepdims=True)
        cnt = jnp.zeros((_QB, 1), dtype=_f32)
        for t in range(_NJ):
            st = simbuf[:, t * _CK:(t + 1) * _CK]
            cnt = cnt + jnp.sum((st >= v).astype(_f32), axis=1, keepdims=True)
        bad = jnp.any((v[:, 0] > -1e8) & (cnt[:, 0] > 16.5))

        @pl.when(bad)
        def _exact_fallback():
            vv = m1
            for _ in range(15):
                nxt = None
                for t in range(_NJ):
                    st = simbuf[:, t * _CK:(t + 1) * _CK]
                    pm = jnp.max(jnp.where(st < vv, st, -jnp.inf), axis=1,
                                 keepdims=True)
                    nxt = pm if nxt is None else jnp.maximum(nxt, pm)
                vv = nxt
            v16_ref[...] = jnp.broadcast_to(vv, (_QB, _LCH))

        @pl.when(jnp.logical_not(bad))
        def _fast():
            v16_ref[...] = jnp.broadcast_to(v, (_QB, _LCH))

        v16 = v16_ref[...][:, 0:1]
        den = jnp.zeros((_QB, 1), dtype=_f32)
        for t in range(_NJ):
            st = simbuf[:, t * _CK:(t + 1) * _CK]
            sel = (st >= v16) & (st > -1e8)
            e = jnp.where(sel, jnp.exp(st - m1), 0.0)
            simbuf[:, pl.ds(t * _CK, _CK)] = e
            den = den + jnp.sum(e, axis=1, keepdims=True)
        den_ref[...] = jnp.broadcast_to(den, (_QB, _LCH))

    @pl.when(j > _NJ)
    def _agg_phase():
        t = j - _NJ - 1
        e = simbuf[:, pl.ds(pl.multiple_of(t * _CK, _CK), _CK)]
        part = jnp.dot(e.astype(_bf16), hb0_ref[...],
                       preferred_element_type=_f32)

        @pl.when(j == _NJ + 1)
        def _():
            acc_ref[...] = part

        @pl.when(j > _NJ + 1)
        def _():
            acc_ref[...] = acc_ref[...] + part

        @pl.when(j == 2 * _NJ)
        def _():
            out_ref[...] = acc_ref[...] / (den_ref[...][:, 0:1] + 1e-9)


def _fuse_kernel(aggt_ref, aggn_ref, aggw_ref, wvt_ref, wvn_ref, wvw_ref,
                 w1a_ref, w1b_ref, w1c_ref, b1_ref, w2_ref, b2_ref, out_ref):
    ot = _bdot(aggt_ref[...], wvt_ref[...])
    on = _bdot(aggn_ref[...], wvn_ref[...])
    ow = _bdot(aggw_ref[...], wvw_ref[...])
    h = (_bdot(ot, w1a_ref[...]) + _bdot(on, w1b_ref[...])
         + _bdot(ow, w1c_ref[...]) + b1_ref[...])
    h = jnp.maximum(h, 0.0)
    out_ref[...] = _bdot(h, w2_ref[...]) + b2_ref[...]


def kernel(current_band_0, current_band_1, current_band_2,
           history_band_0, history_band_1, history_band_2,
           Wq_t, Wk_t, Wv_t, Wq_n, Wk_n, Wv_n, Wv_w,
           fus_W1, fus_b1, fus_W2, fus_b2, current_step):
    f32 = jnp.float32
    cb0f = current_band_0.reshape(_N, _D)
    cb1f = current_band_1.reshape(_N, _D)
    hb0_bf = history_band_0.reshape(_K, _D).astype(_bf16)
    hb0_t = hb0_bf.reshape(_T, _N, _D)
    hb1f = history_band_1.reshape(_K, _D)
    hb2_5 = (history_band_2[_T - _LAYER_RANGE:]
             .reshape(_LAYER_RANGE, _N, _D).astype(_bf16))

    kn = pl.pallas_call(
        _norm_kernel,
        grid=(16,),
        in_specs=[pl.BlockSpec((2048, _D), lambda j: (j, 0))],
        out_specs=pl.BlockSpec((2048, _D), lambda j: (j, 0)),
        out_shape=jax.ShapeDtypeStruct((_K, _D), _bf16),
    )(hb1f)

    full = lambda shape: pl.BlockSpec(shape, lambda: tuple(0 for _ in shape))
    _PB = 128
    aggt, aggn = pl.pallas_call(
        _small_kernel,
        grid=(_N // _PB,),
        in_specs=[pl.BlockSpec((_PB, _D), lambda i: (i, 0)),
                  pl.BlockSpec((_PB, _D), lambda i: (i, 0)),
                  pl.BlockSpec((_T, _PB, _D), lambda i: (0, i, 0)),
                  pl.BlockSpec((_LAYER_RANGE, _PB, _D), lambda i: (0, i, 0)),
                  pl.BlockSpec((_D, _A), lambda i: (0, 0)),
                  pl.BlockSpec((_D, _A), lambda i: (0, 0)),
                  pl.BlockSpec((_D, _A), lambda i: (0, 0)),
                  pl.BlockSpec((_D, _A), lambda i: (0, 0))],
        out_specs=[pl.BlockSpec((_PB, _D), lambda i: (i, 0)),
                   pl.BlockSpec((_PB, _D), lambda i: (i, 0))],
        out_shape=[jax.ShapeDtypeStruct((_N, _D), f32),
                   jax.ShapeDtypeStruct((_N, _D), f32)],
    )(cb0f, cb1f, hb0_t, hb2_5, Wq_t, Wk_t, Wq_n, Wk_n)

    aggw = pl.pallas_call(
        _worm_kernel,
        grid=(_N // _QB, 2 * _NJ + 1),
        in_specs=[pl.BlockSpec((_QB, _D), lambda i, j: (i, 0)),
                  pl.BlockSpec((_CK, _D), lambda i, j: (jnp.minimum(j, _NJ - 1), 0)),
                  pl.BlockSpec((_CK, _D), lambda i, j: (jnp.clip(j - _NJ - 1, 0, _NJ - 1), 0))],
        out_specs=pl.BlockSpec((_QB, _D), lambda i, j: (i, 0)),
        out_shape=jax.ShapeDtypeStruct((_N, _D), f32),
        scratch_shapes=[pltpu.VMEM((_QB, _K), f32),
                        pltpu.VMEM((_QB, _NJ * _LCH), f32),
                        pltpu.VMEM((_QB, _LCH), f32),
                        pltpu.VMEM((_QB, _LCH), f32),
                        pltpu.VMEM((_QB, _D), f32)],
    )(cb1f, kn, hb0_bf)

    out = pl.pallas_call(
        _fuse_kernel,
        grid=(),
        in_specs=[full((_N, _D)), full((_N, _D)), full((_N, _D)),
                  full((_D, _D)), full((_D, _D)), full((_D, _D)),
                  full((_D, _A)), full((_D, _A)), full((_D, _A)),
                  full((1, _A)), full((_A, _A)), full((1, _A))],
        out_specs=full((_N, _A)),
        out_shape=jax.ShapeDtypeStruct((_N, _A), f32),
    )(aggt, aggn, aggw, Wv_t, Wv_n, Wv_w,
      fus_W1[:_D], fus_W1[_D:2 * _D], fus_W1[2 * _D:],
      fus_b1.reshape(1, _A), fus_W2, fus_b2.reshape(1, _A))

    return out.reshape(_H, _W, _A)


# speedup vs baseline: 5.7655x; 1.3583x over previous
import jax
import jax.numpy as jnp
from jax.experimental import pallas as pl
from jax.experimental.pallas import tpu as pltpu

_H, _W, _T, _D, _A = 32, 32, 32, 256, 512
_N = _H * _W
_K = _T * _N
_TOPK_T = 4
_DECAY = 0.95
_LAYER_RANGE = 5
_WH_THRESH = 0.0
_WH_MAXCONN = 16
_TAU = 1.0
_NEG = -1e9

_bf16 = jnp.bfloat16
_f32 = jnp.float32


def _bdot(a, b):
    return jnp.dot(a.astype(_bf16), b.astype(_bf16),
                   preferred_element_type=_f32)


def _bdot_t(a, b):
    return jax.lax.dot_general(a.astype(_bf16), b.astype(_bf16),
                               (((1,), (1,)), ((), ())),
                               preferred_element_type=_f32)


def _norm_kernel(x_ref, o_ref):
    x = x_ref[...]
    n = jnp.sqrt(jnp.sum(x * x, axis=-1, keepdims=True))
    o_ref[...] = (x / (n + 1e-6)).astype(_bf16)


def _small_kernel(cb0_ref, cb1_ref, hb0_ref, hb2_ref,
                  wqt_ref, wkt_ref, wqn_ref, wkn_ref,
                  aggt_ref, aggn_ref):
    scale = 1.0 / jnp.sqrt(jnp.float32(_A))
    cb0 = cb0_ref[...]
    cb1 = cb1_ref[...]
    qt = _bdot(cb0, wqt_ref[...])
    qt_r = qt.astype(_bf16).astype(_f32)
    pb = cb0.shape[0]
    hb0_all = hb0_ref[...].reshape(_T * pb, _D)
    kt_all = _bdot(hb0_all, wkt_ref[...])
    kt_all = kt_all.astype(_bf16).astype(_f32)
    cols = []
    logd = jnp.log(jnp.float32(_DECAY))
    for t in range(_T):
        kt_r = kt_all[t * pb:(t + 1) * pb]
        s = jnp.sum(qt_r * kt_r, axis=-1) * scale + logd * (_T - t)
        cols.append(s[:, None])
    scores = jnp.concatenate(cols, axis=1)
    work = scores
    m1 = None
    vk = None
    for i in range(_TOPK_T):
        vk = jnp.max(work, axis=1, keepdims=True)
        if i == 0:
            m1 = vk
        work = jnp.where(work == vk, -jnp.inf, work)
    e = jnp.where(scores >= vk, jnp.exp((scores - m1) / _TAU), 0.0)
    wt = e / jnp.sum(e, axis=1, keepdims=True)
    agg = wt[:, 0][:, None] * hb0_ref[0]
    for t in range(1, _T):
        agg = agg + wt[:, t][:, None] * hb0_ref[t]
    aggt_ref[...] = agg
    mn = _bdot_t(wqn_ref[...], wkn_ref[...])
    qkn = _bdot(cb1, mn)
    ncols = []
    for t in range(_LAYER_RANGE):
        s = jnp.sum(hb2_ref[t] * qkn, axis=-1) * scale
        ncols.append(s[:, None])
    sn = jnp.concatenate(ncols, axis=1)
    mx = jnp.max(sn, axis=1, keepdims=True)
    en = jnp.exp((sn - mx) / _TAU)
    wn = en / jnp.sum(en, axis=1, keepdims=True)
    aggn = wn[:, 0][:, None] * hb2_ref[0]
    for t in range(1, _LAYER_RANGE):
        aggn = aggn + wn[:, t][:, None] * hb2_ref[t]
    aggn_ref[...] = aggn


_QB = 256
_CK = 4096
_NJ = _K // _CK
_LCH = 128
_NCH = _CK // _LCH
_TOP_PER_CH = 4
_CW = _TOP_PER_CH * _LCH


def _worm_kernel(cb1_ref, kn_ref, hb0_ref, out_ref, simbuf, cand_ref, v16_ref,
                 den_ref, acc_ref):
    j = pl.program_id(1)

    @pl.when(j < _NJ)
    def _sim_phase():
        q = cb1_ref[...]
        qn = (q / (jnp.sqrt(jnp.sum(q * q, axis=-1, keepdims=True)) + 1e-6)
              ).astype(_bf16)
        s = jax.lax.dot_general(qn, kn_ref[...], (((1,), (1,)), ((), ())),
                                preferred_element_type=_f32)
        sm = jnp.where(s >= 0.0, s, _NEG)
        simbuf[:, pl.ds(pl.multiple_of(j * _CK, _CK), _CK)] = sm
        slices = [sm[:, k * _LCH:(k + 1) * _LCH] for k in range(_NCH)]
        g = slices[0]
        for s_ in slices[1:]:
            g = jnp.maximum(g, s_)
        ranks = [g]
        work = slices
        for _ in range(1, _TOP_PER_CH):
            work = [jnp.where(s_ < g, s_, -jnp.inf) for s_ in work]
            g = work[0]
            for s_ in work[1:]:
                g = jnp.maximum(g, s_)
            ranks.append(g)
        block = jnp.concatenate(ranks, axis=1)
        cand_ref[:, pl.ds(pl.multiple_of(j * _CW, _LCH), _CW)] = block

    @pl.when(j == _NJ)
    def _select_phase():
        cand = cand_ref[...]
        m1 = jnp.max(cand, axis=1, keepdims=True)
        v = m1
        for _ in range(15):
            v = jnp.max(jnp.where(cand < v, cand, -jnp.inf), axis=1,
                        keepdims=True)
        cnt = jnp.zeros((_QB, 1), dtype=_f32)
        for t in range(_NJ):
            st = simbuf[:, t * _CK:(t + 1) * _CK]
            cnt = cnt + jnp.sum((st >= v).astype(_f32), axis=1, keepdims=True)
        bad = jnp.any((v[:, 0] > -1e8) & (cnt[:, 0] > 16.5))

        @pl.when(bad)
        def _exact_fallback():
            vv = m1
            for _ in range(15):
                nxt = None
                for t in range(_NJ):
                    st = simbuf[:, t * _CK:(t + 1) * _CK]
                    pm = jnp.max(jnp.where(st < vv, st, -jnp.inf), axis=1,
                                 keepdims=True)
                    nxt = pm if nxt is None else jnp.maximum(nxt, pm)
                vv = nxt
            v16_ref[...] = jnp.broadcast_to(vv, (_QB, _LCH))

        @pl.when(jnp.logical_not(bad))
        def _fast():
            v16_ref[...] = jnp.broadcast_to(v, (_QB, _LCH))

        v16 = v16_ref[...][:, 0:1]
        den = jnp.zeros((_QB, 1), dtype=_f32)
        for t in range(_NJ):
            st = simbuf[:, t * _CK:(t + 1) * _CK]
            sel = (st >= v16) & (st > -1e8)
            e = jnp.where(sel, jnp.exp(st - m1), 0.0)
            simbuf[:, pl.ds(t * _CK, _CK)] = e
            den = den + jnp.sum(e, axis=1, keepdims=True)
        den_ref[...] = jnp.broadcast_to(den, (_QB, _LCH))

    @pl.when(j > _NJ)
    def _agg_phase():
        t = j - _NJ - 1
        e = simbuf[:, pl.ds(pl.multiple_of(t * _CK, _CK), _CK)]
        part = jnp.dot(e.astype(_bf16), hb0_ref[...],
                       preferred_element_type=_f32)

        @pl.when(j == _NJ + 1)
        def _():
            acc_ref[...] = part

        @pl.when(j > _NJ + 1)
        def _():
            acc_ref[...] = acc_ref[...] + part

        @pl.when(j == 2 * _NJ)
        def _():
            out_ref[...] = acc_ref[...] / (den_ref[...][:, 0:1] + 1e-9)


def _fuse_kernel(aggt_ref, aggn_ref, aggw_ref, wvt_ref, wvn_ref, wvw_ref,
                 w1a_ref, w1b_ref, w1c_ref, b1_ref, w2_ref, b2_ref, out_ref):
    ot = _bdot(aggt_ref[...], wvt_ref[...])
    on = _bdot(aggn_ref[...], wvn_ref[...])
    ow = _bdot(aggw_ref[...], wvw_ref[...])
    h = (_bdot(ot, w1a_ref[...]) + _bdot(on, w1b_ref[...])
         + _bdot(ow, w1c_ref[...]) + b1_ref[...])
    h = jnp.maximum(h, 0.0)
    out_ref[...] = _bdot(h, w2_ref[...]) + b2_ref[...]


def kernel(current_band_0, current_band_1, current_band_2,
           history_band_0, history_band_1, history_band_2,
           Wq_t, Wk_t, Wv_t, Wq_n, Wk_n, Wv_n, Wv_w,
           fus_W1, fus_b1, fus_W2, fus_b2, current_step):
    f32 = jnp.float32
    cb0f = current_band_0.reshape(_N, _D)
    cb1f = current_band_1.reshape(_N, _D)
    hb0_bf = history_band_0.reshape(_K, _D).astype(_bf16)
    hb0_t = hb0_bf.reshape(_T, _N, _D)
    hb1f = history_band_1.reshape(_K, _D)
    hb2_5 = (history_band_2[_T - _LAYER_RANGE:]
             .reshape(_LAYER_RANGE, _N, _D).astype(_bf16))

    kn = pl.pallas_call(
        _norm_kernel,
        grid=(16,),
        in_specs=[pl.BlockSpec((2048, _D), lambda j: (j, 0))],
        out_specs=pl.BlockSpec((2048, _D), lambda j: (j, 0)),
        out_shape=jax.ShapeDtypeStruct((_K, _D), _bf16),
    )(hb1f)

    full = lambda shape: pl.BlockSpec(shape, lambda: tuple(0 for _ in shape))
    _PB = 128
    aggt, aggn = pl.pallas_call(
        _small_kernel,
        grid=(_N // _PB,),
        in_specs=[pl.BlockSpec((_PB, _D), lambda i: (i, 0)),
                  pl.BlockSpec((_PB, _D), lambda i: (i, 0)),
                  pl.BlockSpec((_T, _PB, _D), lambda i: (0, i, 0)),
                  pl.BlockSpec((_LAYER_RANGE, _PB, _D), lambda i: (0, i, 0)),
                  pl.BlockSpec((_D, _A), lambda i: (0, 0)),
                  pl.BlockSpec((_D, _A), lambda i: (0, 0)),
                  pl.BlockSpec((_D, _A), lambda i: (0, 0)),
                  pl.BlockSpec((_D, _A), lambda i: (0, 0))],
        out_specs=[pl.BlockSpec((_PB, _D), lambda i: (i, 0)),
                   pl.BlockSpec((_PB, _D), lambda i: (i, 0))],
        out_shape=[jax.ShapeDtypeStruct((_N, _D), f32),
                   jax.ShapeDtypeStruct((_N, _D), f32)],
    )(cb0f, cb1f, hb0_t, hb2_5, Wq_t, Wk_t, Wq_n, Wk_n)

    aggw = pl.pallas_call(
        _worm_kernel,
        grid=(_N // _QB, 2 * _NJ + 1),
        in_specs=[pl.BlockSpec((_QB, _D), lambda i, j: (i, 0)),
                  pl.BlockSpec((_CK, _D), lambda i, j: (jnp.minimum(j, _NJ - 1), 0)),
                  pl.BlockSpec((_CK, _D), lambda i, j: (jnp.clip(j - _NJ - 1, 0, _NJ - 1), 0))],
        out_specs=pl.BlockSpec((_QB, _D), lambda i, j: (i, 0)),
        out_shape=jax.ShapeDtypeStruct((_N, _D), f32),
        scratch_shapes=[pltpu.VMEM((_QB, _K), f32),
                        pltpu.VMEM((_QB, _NJ * _CW), f32),
                        pltpu.VMEM((_QB, _LCH), f32),
                        pltpu.VMEM((_QB, _LCH), f32),
                        pltpu.VMEM((_QB, _D), f32)],
    )(cb1f, kn, hb0_bf)

    out = pl.pallas_call(
        _fuse_kernel,
        grid=(),
        in_specs=[full((_N, _D)), full((_N, _D)), full((_N, _D)),
                  full((_D, _D)), full((_D, _D)), full((_D, _D)),
                  full((_D, _A)), full((_D, _A)), full((_D, _A)),
                  full((1, _A)), full((_A, _A)), full((1, _A))],
        out_specs=full((_N, _A)),
        out_shape=jax.ShapeDtypeStruct((_N, _A), f32),
    )(aggt, aggn, aggw, Wv_t, Wv_n, Wv_w,
      fus_W1[:_D], fus_W1[_D:2 * _D], fus_W1[2 * _D:],
      fus_b1.reshape(1, _A), fus_W2, fus_b2.reshape(1, _A))

    return out.reshape(_H, _W, _A)


# speedup vs baseline: 7.0238x; 1.2182x over previous
import jax
import jax.numpy as jnp
from jax.experimental import pallas as pl
from jax.experimental.pallas import tpu as pltpu

_H, _W, _T, _D, _A = 32, 32, 32, 256, 512
_N = _H * _W
_K = _T * _N
_TOPK_T = 4
_DECAY = 0.95
_LAYER_RANGE = 5
_WH_THRESH = 0.0
_WH_MAXCONN = 16
_TAU = 1.0
_NEG = -1e9

_bf16 = jnp.bfloat16
_f32 = jnp.float32


def _bdot(a, b):
    return jnp.dot(a.astype(_bf16), b.astype(_bf16),
                   preferred_element_type=_f32)


def _bdot_t(a, b):
    return jax.lax.dot_general(a.astype(_bf16), b.astype(_bf16),
                               (((1,), (1,)), ((), ())),
                               preferred_element_type=_f32)


def _norm_kernel(x_ref, h0_ref, o_ref, h0b_ref):
    x = x_ref[...]
    n = jnp.sqrt(jnp.sum(x * x, axis=-1, keepdims=True))
    o_ref[...] = (x / (n + 1e-6)).astype(_bf16)
    h0b_ref[...] = h0_ref[...].astype(_bf16)


def _small_kernel(cb0_ref, cb1_ref, hb0_ref, hb2_ref,
                  wqt_ref, wkt_ref, wqn_ref, wkn_ref,
                  aggt_ref, aggn_ref):
    scale = 1.0 / jnp.sqrt(jnp.float32(_A))
    cb0 = cb0_ref[...]
    cb1 = cb1_ref[...]
    qt = _bdot(cb0, wqt_ref[...])
    qt_r = qt.astype(_bf16).astype(_f32)
    pb = cb0.shape[0]
    hb0_all = hb0_ref[...].reshape(_T * pb, _D)
    kt_all = _bdot(hb0_all, wkt_ref[...])
    kt_all = kt_all.astype(_bf16).astype(_f32)
    cols = []
    logd = jnp.log(jnp.float32(_DECAY))
    for t in range(_T):
        kt_r = kt_all[t * pb:(t + 1) * pb]
        s = jnp.sum(qt_r * kt_r, axis=-1) * scale + logd * (_T - t)
        cols.append(s[:, None])
    scores = jnp.concatenate(cols, axis=1)
    work = scores
    m1 = None
    vk = None
    for i in range(_TOPK_T):
        vk = jnp.max(work, axis=1, keepdims=True)
        if i == 0:
            m1 = vk
        work = jnp.where(work == vk, -jnp.inf, work)
    e = jnp.where(scores >= vk, jnp.exp((scores - m1) / _TAU), 0.0)
    wt = e / jnp.sum(e, axis=1, keepdims=True)
    agg = wt[:, 0][:, None] * hb0_ref[0]
    for t in range(1, _T):
        agg = agg + wt[:, t][:, None] * hb0_ref[t]
    aggt_ref[...] = agg
    mn = _bdot_t(wqn_ref[...], wkn_ref[...])
    qkn = _bdot(cb1, mn)
    ncols = []
    for t in range(_LAYER_RANGE):
        s = jnp.sum(hb2_ref[t] * qkn, axis=-1) * scale
        ncols.append(s[:, None])
    sn = jnp.concatenate(ncols, axis=1)
    mx = jnp.max(sn, axis=1, keepdims=True)
    en = jnp.exp((sn - mx) / _TAU)
    wn = en / jnp.sum(en, axis=1, keepdims=True)
    aggn = wn[:, 0][:, None] * hb2_ref[0]
    for t in range(1, _LAYER_RANGE):
        aggn = aggn + wn[:, t][:, None] * hb2_ref[t]
    aggn_ref[...] = aggn


_QB = 256
_CK = 4096
_NJ = _K // _CK
_LCH = 128
_NCH = _CK // _LCH
_TOP_PER_CH = 4
_CW = _TOP_PER_CH * _LCH


def _worm_kernel(cb1_ref, kn_ref, hb0_ref, out_ref, simbuf, cand_ref, v16_ref,
                 den_ref, acc_ref, acc2_ref):
    j = pl.program_id(1)

    @pl.when(j < _NJ)
    def _sim_phase():
        q = cb1_ref[...]
        qn = (q / (jnp.sqrt(jnp.sum(q * q, axis=-1, keepdims=True)) + 1e-6)
              ).astype(_bf16)
        s = jax.lax.dot_general(qn, kn_ref[...], (((1,), (1,)), ((), ())),
                                preferred_element_type=_f32)
        sm = jnp.where(s >= 0.0, s, _NEG)
        simbuf[:, pl.ds(pl.multiple_of(j * _CK, _CK), _CK)] = sm
        slices = [sm[:, k * _LCH:(k + 1) * _LCH] for k in range(_NCH)]
        g = slices[0]
        for s_ in slices[1:]:
            g = jnp.maximum(g, s_)
        ranks = [g]
        work = slices
        for _ in range(1, _TOP_PER_CH):
            work = [jnp.where(s_ < g, s_, -jnp.inf) for s_ in work]
            g = work[0]
            for s_ in work[1:]:
                g = jnp.maximum(g, s_)
            ranks.append(g)
        block = jnp.concatenate(ranks, axis=1)
        cand_ref[:, pl.ds(pl.multiple_of(j * _CW, _LCH), _CW)] = block

    @pl.when(j == _NJ)
    def _select_phase():
        cand = cand_ref[...]
        m1 = jnp.max(cand, axis=1, keepdims=True)
        v = m1
        for _ in range(15):
            v = jnp.max(jnp.where(cand < v, cand, -jnp.inf), axis=1,
                        keepdims=True)
        cnt = jnp.zeros((_QB, 1), dtype=_f32)
        for t in range(_NJ):
            st = simbuf[:, t * _CK:(t + 1) * _CK]
            cnt = cnt + jnp.sum((st >= v).astype(_f32), axis=1, keepdims=True)
        bad = jnp.any((v[:, 0] > -1e8) & (cnt[:, 0] > 16.5))

        @pl.when(bad)
        def _exact_fallback():
            vv = m1
            for _ in range(15):
                nxt = None
                for t in range(_NJ):
                    st = simbuf[:, t * _CK:(t + 1) * _CK]
                    pm = jnp.max(jnp.where(st < vv, st, -jnp.inf), axis=1,
                                 keepdims=True)
                    nxt = pm if nxt is None else jnp.maximum(nxt, pm)
                vv = nxt
            v16_ref[...] = jnp.broadcast_to(vv, (_QB, _LCH))

        @pl.when(jnp.logical_not(bad))
        def _fast():
            v16_ref[...] = jnp.broadcast_to(v, (_QB, _LCH))

        den_ref[...] = jnp.broadcast_to(m1, (_QB, _LCH))

    @pl.when(j > _NJ)
    def _agg_phase():
        t = j - _NJ - 1
        st = simbuf[:, pl.ds(pl.multiple_of(t * _CK, _CK), _CK)]
        v16 = v16_ref[...][:, 0:1]
        m1 = den_ref[...][:, 0:1]
        sel = (st >= v16) & (st > -1e8)
        e = jnp.where(sel, jnp.exp(st - m1), 0.0)
        part = jnp.dot(e.astype(_bf16), hb0_ref[...],
                       preferred_element_type=_f32)
        dpart = jnp.sum(e, axis=1, keepdims=True)

        @pl.when(j == _NJ + 1)
        def _():
            acc_ref[...] = part
            acc2_ref[...] = jnp.broadcast_to(dpart, (_QB, _LCH))

        @pl.when(j > _NJ + 1)
        def _():
            acc_ref[...] = acc_ref[...] + part
            acc2_ref[...] = acc2_ref[...] + jnp.broadcast_to(dpart,
                                                            (_QB, _LCH))

        @pl.when(j == 2 * _NJ)
        def _():
            out_ref[...] = acc_ref[...] / (acc2_ref[...][:, 0:1] + 1e-9)


def _fuse_kernel(aggt_ref, aggn_ref, aggw_ref, wvt_ref, wvn_ref, wvw_ref,
                 w1a_ref, w1b_ref, w1c_ref, b1_ref, w2_ref, b2_ref, out_ref):
    ot = _bdot(aggt_ref[...], wvt_ref[...])
    on = _bdot(aggn_ref[...], wvn_ref[...])
    ow = _bdot(aggw_ref[...], wvw_ref[...])
    h = (_bdot(ot, w1a_ref[...]) + _bdot(on, w1b_ref[...])
         + _bdot(ow, w1c_ref[...]) + b1_ref[...])
    h = jnp.maximum(h, 0.0)
    out_ref[...] = _bdot(h, w2_ref[...]) + b2_ref[...]


def kernel(current_band_0, current_band_1, current_band_2,
           history_band_0, history_band_1, history_band_2,
           Wq_t, Wk_t, Wv_t, Wq_n, Wk_n, Wv_n, Wv_w,
           fus_W1, fus_b1, fus_W2, fus_b2, current_step):
    f32 = jnp.float32
    cb0f = current_band_0.reshape(_N, _D)
    cb1f = current_band_1.reshape(_N, _D)
    hb0f = history_band_0.reshape(_K, _D)
    hb1f = history_band_1.reshape(_K, _D)
    hb2_5 = (history_band_2[_T - _LAYER_RANGE:]
             .reshape(_LAYER_RANGE, _N, _D).astype(_bf16))

    kn, hb0_bf = pl.pallas_call(
        _norm_kernel,
        grid=(16,),
        in_specs=[pl.BlockSpec((2048, _D), lambda j: (j, 0)),
                  pl.BlockSpec((2048, _D), lambda j: (j, 0))],
        out_specs=[pl.BlockSpec((2048, _D), lambda j: (j, 0)),
                   pl.BlockSpec((2048, _D), lambda j: (j, 0))],
        out_shape=[jax.ShapeDtypeStruct((_K, _D), _bf16),
                   jax.ShapeDtypeStruct((_K, _D), _bf16)],
    )(hb1f, hb0f)
    hb0_t = hb0_bf.reshape(_T, _N, _D)

    full = lambda shape: pl.BlockSpec(shape, lambda: tuple(0 for _ in shape))
    _PB = 128
    aggt, aggn = pl.pallas_call(
        _small_kernel,
        grid=(_N // _PB,),
        in_specs=[pl.BlockSpec((_PB, _D), lambda i: (i, 0)),
                  pl.BlockSpec((_PB, _D), lambda i: (i, 0)),
                  pl.BlockSpec((_T, _PB, _D), lambda i: (0, i, 0)),
                  pl.BlockSpec((_LAYER_RANGE, _PB, _D), lambda i: (0, i, 0)),
                  pl.BlockSpec((_D, _A), lambda i: (0, 0)),
                  pl.BlockSpec((_D, _A), lambda i: (0, 0)),
                  pl.BlockSpec((_D, _A), lambda i: (0, 0)),
                  pl.BlockSpec((_D, _A), lambda i: (0, 0))],
        out_specs=[pl.BlockSpec((_PB, _D), lambda i: (i, 0)),
                   pl.BlockSpec((_PB, _D), lambda i: (i, 0))],
        out_shape=[jax.ShapeDtypeStruct((_N, _D), f32),
                   jax.ShapeDtypeStruct((_N, _D), f32)],
    )(cb0f, cb1f, hb0_t, hb2_5, Wq_t, Wk_t, Wq_n, Wk_n)

    aggw = pl.pallas_call(
        _worm_kernel,
        grid=(_N // _QB, 2 * _NJ + 1),
        in_specs=[pl.BlockSpec((_QB, _D), lambda i, j: (i, 0)),
                  pl.BlockSpec((_CK, _D), lambda i, j: (jnp.minimum(j, _NJ - 1), 0)),
                  pl.BlockSpec((_CK, _D), lambda i, j: (jnp.clip(j - _NJ - 1, 0, _NJ - 1), 0))],
        out_specs=pl.BlockSpec((_QB, _D), lambda i, j: (i, 0)),
        out_shape=jax.ShapeDtypeStruct((_N, _D), f32),
        scratch_shapes=[pltpu.VMEM((_QB, _K), f32),
                        pltpu.VMEM((_QB, _NJ * _CW), f32),
                        pltpu.VMEM((_QB, _LCH), f32),
                        pltpu.VMEM((_QB, _LCH), f32),
                        pltpu.VMEM((_QB, _D), f32),
                        pltpu.VMEM((_QB, _LCH), f32)],
    )(cb1f, kn, hb0_bf)

    out = pl.pallas_call(
        _fuse_kernel,
        grid=(),
        in_specs=[full((_N, _D)), full((_N, _D)), full((_N, _D)),
                  full((_D, _D)), full((_D, _D)), full((_D, _D)),
                  full((_D, _A)), full((_D, _A)), full((_D, _A)),
                  full((1, _A)), full((_A, _A)), full((1, _A))],
        out_specs=full((_N, _A)),
        out_shape=jax.ShapeDtypeStruct((_N, _A), f32),
    )(aggt, aggn, aggw, Wv_t, Wv_n, Wv_w,
      fus_W1[:_D], fus_W1[_D:2 * _D], fus_W1[2 * _D:],
      fus_b1.reshape(1, _A), fus_W2, fus_b2.reshape(1, _A))

    return out.reshape(_H, _W, _A)
